# baseline (speedup 1.0000x reference)
import functools

import jax
import jax.numpy as jnp
from jax import lax
from jax.experimental import pallas as pl
from jax.experimental.pallas import tpu as pltpu
from jax.experimental.pallas import tpu_sc as plsc

_N = 16777216
_NC = 2
_NS = 16
_NW = _NC * _NS
_PER_W = _N // _NW
_CHUNK = 16384
_NCHUNK = _PER_W // _CHUNK

_mesh = plsc.VectorSubcoreMesh(core_axis_name="c", subcore_axis_name="s")


@functools.partial(
    pl.kernel,
    mesh=_mesh,
    out_type=jax.ShapeDtypeStruct((_N,), jnp.float32),
    scratch_types=[pltpu.VMEM((_CHUNK,), jnp.float32)],
)
def _sc_binarize(x_hbm, out_hbm, buf):
    wid = lax.axis_index("s") * _NC + lax.axis_index("c")
    base = wid * _PER_W

    def chunk_body(ci, carry):
        off = base + ci * _CHUNK
        pltpu.sync_copy(x_hbm.at[pl.ds(off, _CHUNK)], buf)

        def vec_body(vi, c):
            v = buf[pl.ds(vi * 16, 16)]
            buf[pl.ds(vi * 16, 16)] = jnp.where(
                v > 1.0, jnp.float32(1.0), jnp.float32(0.0)
            )
            return c

        lax.fori_loop(0, _CHUNK // 16, vec_body, 0)
        pltpu.sync_copy(buf, out_hbm.at[pl.ds(off, _CHUNK)])
        return carry

    lax.fori_loop(0, _NCHUNK, chunk_body, 0)


def kernel(x):
    return _sc_binarize(x)


# speedup vs baseline: 3.4861x; 3.4861x over previous
import functools

import jax
import jax.numpy as jnp
from jax import lax
from jax.experimental import pallas as pl
from jax.experimental.pallas import tpu as pltpu
from jax.experimental.pallas import tpu_sc as plsc

_N = 16777216
_NC = 2
_NS = 16
_NW = _NC * _NS
_PER_W = _N // _NW
_CHUNK = 32768
_NCHUNK = _PER_W // _CHUNK
_UNROLL = 8

_mesh = plsc.VectorSubcoreMesh(core_axis_name="c", subcore_axis_name="s")


def _binarize_buf(buf):
    def vec_body(vi, c):
        b = vi * (16 * _UNROLL)
        for u in range(_UNROLL):
            o = b + u * 16
            v = buf[pl.ds(o, 16)]
            buf[pl.ds(o, 16)] = jnp.where(
                v > 1.0, jnp.float32(1.0), jnp.float32(0.0)
            )
        return c

    lax.fori_loop(0, _CHUNK // (16 * _UNROLL), vec_body, 0)


@functools.partial(
    pl.kernel,
    mesh=_mesh,
    out_type=jax.ShapeDtypeStruct((_N,), jnp.float32),
    scratch_types=[
        pltpu.VMEM((_CHUNK,), jnp.float32),
        pltpu.VMEM((_CHUNK,), jnp.float32),
        pltpu.SemaphoreType.DMA,
        pltpu.SemaphoreType.DMA,
        pltpu.SemaphoreType.DMA,
        pltpu.SemaphoreType.DMA,
    ],
)
def _sc_binarize(x_hbm, out_hbm, buf0, buf1, ld0, ld1, st0, st1):
    wid = lax.axis_index("s") * _NC + lax.axis_index("c")
    base = wid * _PER_W

    bufs = (buf0, buf1)
    ldsems = (ld0, ld1)
    stsems = (st0, st1)

    loads = [None, None]
    stores = [None, None]

    loads[0] = pltpu.async_copy(x_hbm.at[pl.ds(base, _CHUNK)], bufs[0], ldsems[0])
    for ci in range(_NCHUNK):
        cur = ci % 2
        nxt = (ci + 1) % 2
        if ci + 1 < _NCHUNK:
            if stores[nxt] is not None:
                stores[nxt].wait()
            loads[nxt] = pltpu.async_copy(
                x_hbm.at[pl.ds(base + (ci + 1) * _CHUNK, _CHUNK)],
                bufs[nxt],
                ldsems[nxt],
            )
        loads[cur].wait()
        _binarize_buf(bufs[cur])
        stores[cur] = pltpu.async_copy(
            bufs[cur], out_hbm.at[pl.ds(base + ci * _CHUNK, _CHUNK)], stsems[cur]
        )
    stores[0].wait()
    stores[1].wait()


def kernel(x):
    return _sc_binarize(x)
